# baseline (speedup 1.0000x reference)
import jax
import jax.numpy as jnp
from jax import lax
from jax.experimental import pallas as pl
from jax.experimental.pallas import tpu as pltpu
from jax.experimental.pallas import tpu_sc as plsc

_NC, _NS, _L = 2, 16, 16
_NW = _NC * _NS
_P = 128
_PLANE_DIMID = ((0, 1), (0, 2), (1, 2))
_NT = 10


def _make_relayout_kernel(C, sizes, interpret=False):
    U = 1024
    E = _L
    counts = [n // (U * _NW) for n in sizes]
    assert all(n % (U * _NW) == 0 for n in sizes)

    mesh = plsc.VectorSubcoreMesh(core_axis_name="c", subcore_axis_name="s",
                                  num_cores=_NC, num_subcores=_NS)
    out_type = tuple(jax.ShapeDtypeStruct((n * 2 * C,), jnp.float32)
                     for n in sizes)
    scratch = [
        pltpu.VMEM((C, U + E), jnp.float32),
        pltpu.VMEM((C, U + E), jnp.float32),
        pltpu.VMEM((U * 2 * C,), jnp.float32),
        pltpu.SemaphoreType.DMA,
        pltpu.SemaphoreType.DMA,
    ]

    def body(*refs):
        n = len(sizes)
        ins = refs[:n]
        outs = refs[n:2 * n]
        inb0, inb1, stg, sem0, sem1 = refs[2 * n:]
        inbs = (inb0, inb1)
        sems = (sem0, sem1)
        wid = lax.axis_index("s") * _NC + lax.axis_index("c")
        lane = lax.iota(jnp.int32, _L)
        lane32 = lane * (2 * C)

        def do_table(t_in, t_out, nunits, ntot):
            ubase = wid * nunits

            def in_descs(i, par, make):
                off = (ubase + i) * U
                full = off + U + E <= ntot

                @pl.when(full)
                def _full():
                    for ch in range(C):
                        make(t_in.at[ch, pl.ds(off, U + E)], inbs[par].at[ch],
                             sems[par])

                @pl.when(jnp.logical_not(full))
                def _short():
                    for ch in range(C):
                        make(t_in.at[ch, pl.ds(off, U)],
                             inbs[par].at[ch, pl.ds(0, U)], sems[par])

            def fire_in(i, par):
                in_descs(i, par, pltpu.async_copy)

            def wait_in(i, par):
                in_descs(i, par,
                         lambda s, d, m: pltpu.make_async_copy(s, d, m).wait())

            def process(i, par):
                inb = inbs[par]

                def grp(i16, cc):
                    nidx = lane32 + i16 * (_L * 2 * C)
                    sl = pl.ds(i16 * _L, _L)
                    for ch in range(C):
                        plsc.store_scatter(stg, [nidx + ch], inb[ch, sl])
                        v1 = plsc.load_gather(inb, [jnp.full((_L,), ch,
                                                             jnp.int32),
                                               lane + (i16 * _L + 1)])
                        plsc.store_scatter(stg, [nidx + (C + ch)], v1)
                    return cc

                lax.fori_loop(0, U // _L, grp, 0)
                off = (ubase + i) * U
                pltpu.sync_copy(stg, t_out.at[pl.ds(off * 2 * C, U * 2 * C)])

            fire_in(0, 0)

            def pairj(j, c):
                i0 = 2 * j
                fire_in(i0 + 1, 1)
                wait_in(i0, 0)
                process(i0, 0)

                @pl.when(i0 + 2 < nunits)
                def _next():
                    fire_in(i0 + 2, 0)

                wait_in(i0 + 1, 1)
                process(i0 + 1, 1)
                return c

            if nunits % 2 == 0:
                lax.fori_loop(0, nunits // 2, pairj, 0)
            else:
                raise NotImplementedError

        for t_in, t_out, cnt, ntot in zip(ins, outs, counts, sizes):
            do_table(t_in, t_out, cnt, ntot)

    return pl.kernel(body, out_type=out_type, mesh=mesh,
                     scratch_types=scratch, interpret=interpret,
                     compiler_params=pltpu.CompilerParams(
                         needs_layout_passes=False,
                         use_tc_tiling_on_sc=False))


def _make_sc_kernel(B, C, vol_dims, pdims, interpret=False):
    D, Hg, Wg = vol_dims
    C2 = 2 * C
    ppt = B // _NW
    P = min(_P, ppt)
    nchunks = ppt // P
    ngroups = P // _L
    pipelined = nchunks % 2 == 0 and nchunks >= 2

    mesh = plsc.VectorSubcoreMesh(core_axis_name="c", subcore_axis_name="s",
                                  num_cores=_NC, num_subcores=_NS)
    out_type = jax.ShapeDtypeStruct((C, B), jnp.float32)
    scratch = [
        pltpu.VMEM((P * 6,), jnp.float32),
        pltpu.VMEM((_NT, P), jnp.int32),
        pltpu.VMEM((_NT, P), jnp.int32),
        pltpu.VMEM((2 * _NT, P), jnp.float32),
        pltpu.VMEM((2 * _NT, P), jnp.float32),
        pltpu.VMEM((_NT, P, C2), jnp.float32),
        pltpu.VMEM((_NT, P, C2), jnp.float32),
        pltpu.VMEM((C, P), jnp.float32),
        pltpu.SemaphoreType.DMA,
        pltpu.SemaphoreType.DMA,
    ]

    def body(xf, g3, p0, p1, p2, out, xv, idx0, idx1, w0, w1, rows0, rows1,
             outv, sem0, sem1):
        wid = lax.axis_index("s") * _NC + lax.axis_index("c")
        tile_base = wid * ppt
        lane = lax.iota(jnp.int32, _L)
        tabs = (g3,) * 4 + (p0,) * 2 + (p1,) * 2 + (p2,) * 2
        idxs = (idx0, idx1)
        rows = (rows0, rows1)
        sems = (sem0, sem1)

        def make_prep(idx_r, w_r):
            def prep_group(g, c):
                sl = pl.ds(g * _L, _L)
                p6 = (lane + g * _L) * 6
                gx = plsc.load_gather(xv, [p6])
                gy = plsc.load_gather(xv, [p6 + 1])
                gz = plsc.load_gather(xv, [p6 + 2])
                coords = (gx, gy, gz)
                ix = (gx + 1.0) * 0.5 * (Wg - 1)
                iy = (gy + 1.0) * 0.5 * (Hg - 1)
                iz = (gz + 1.0) * 0.5 * (D - 1)
                xi = jnp.clip(ix.astype(jnp.int32), 0, Wg - 1)
                yi = jnp.clip(iy.astype(jnp.int32), 0, Hg - 1)
                zi = jnp.clip(iz.astype(jnp.int32), 0, D - 1)
                fx = ix - xi.astype(jnp.float32)
                fy = iy - yi.astype(jnp.float32)
                fz = iz - zi.astype(jnp.float32)
                dy = (jnp.minimum(yi + 1, Hg - 1) - yi) * Wg
                dz = (jnp.minimum(zi + 1, D - 1) - zi) * (Wg * Hg)
                base3 = (zi * Hg + yi) * Wg + xi
                idx_r[0, sl] = base3
                idx_r[1, sl] = base3 + dy
                idx_r[2, sl] = base3 + dz
                idx_r[3, sl] = base3 + dz + dy
                ox = 1.0 - fx
                oy = 1.0 - fy
                oz = 1.0 - fz
                wzy = (oz * oy, oz * fy, fz * oy, fz * fy)
                for r in range(4):
                    w_r[2 * r, sl] = wzy[r] * ox
                    w_r[2 * r + 1, sl] = wzy[r] * fx
                for k, (a, b) in enumerate(_PLANE_DIMID):
                    PHk, PWk = pdims[k]
                    gu = coords[a]
                    gv = coords[b]
                    iu = (gu + 1.0) * 0.5 * (PWk - 1)
                    iv = (gv + 1.0) * 0.5 * (PHk - 1)
                    ui = jnp.clip(iu.astype(jnp.int32), 0, PWk - 1)
                    vi = jnp.clip(iv.astype(jnp.int32), 0, PHk - 1)
                    fu = iu - ui.astype(jnp.float32)
                    fv = iv - vi.astype(jnp.float32)
                    dv = (jnp.minimum(vi + 1, PHk - 1) - vi) * PWk
                    r0 = vi * PWk + ui
                    t0 = 4 + 2 * k
                    idx_r[t0, sl] = r0
                    idx_r[t0 + 1, sl] = r0 + dv
                    ou = 1.0 - fu
                    ov = 1.0 - fv
                    w_r[8 + 4 * k + 0, sl] = ov * ou
                    w_r[8 + 4 * k + 1, sl] = ov * fu
                    w_r[8 + 4 * k + 2, sl] = fv * ou
                    w_r[8 + 4 * k + 3, sl] = fv * fu
                return c
            return prep_group

        def make_combine(rows_r, w_r):
            def combine_group(g, c):
                sl = pl.ds(g * _L, _L)
                pvec = lane + g * _L
                wvs = [w_r[t, sl] for t in range(2 * _NT)]
                for f in range(C):
                    fvec = jnp.full((_L,), f, jnp.int32)
                    fvec2 = jnp.full((_L,), f + C, jnp.int32)
                    a = None
                    for t in range(4):
                        tv = jnp.full((_L,), t, jnp.int32)
                        v0 = plsc.load_gather(rows_r, [tv, pvec, fvec])
                        v1 = plsc.load_gather(rows_r, [tv, pvec, fvec2])
                        u = wvs[2 * t] * v0 + wvs[2 * t + 1] * v1
                        a = u if a is None else a + u
                    for k in range(3):
                        m = None
                        for h in range(2):
                            t = 4 + 2 * k + h
                            tv = jnp.full((_L,), t, jnp.int32)
                            v0 = plsc.load_gather(rows_r, [tv, pvec, fvec])
                            v1 = plsc.load_gather(rows_r, [tv, pvec, fvec2])
                            u = (wvs[8 + 4 * k + 2 * h] * v0
                                 + wvs[8 + 4 * k + 2 * h + 1] * v1)
                            m = u if m is None else m + u
                        a = a * m
                    outv[f, sl] = a
                return c
            return combine_group

        preps = (make_prep(idx0, w0), make_prep(idx1, w1))
        combines = (make_combine(rows0, w0), make_combine(rows1, w1))

        def stage_prep_fire(cc, par):
            base = tile_base + cc * P
            pltpu.sync_copy(xf.at[pl.ds(base * 6, P * 6)], xv)
            lax.fori_loop(0, ngroups, preps[par], 0)
            for t in range(_NT):
                pltpu.async_copy(tabs[t].at[idxs[par].at[t]], rows[par].at[t],
                                 sems[par])

        def wait_combine_store(cc, par):
            for t in range(_NT):
                pltpu.make_async_copy(tabs[t].at[idxs[par].at[t]],
                                      rows[par].at[t], sems[par]).wait()
            lax.fori_loop(0, ngroups, combines[par], 0)
            base = tile_base + cc * P
            pltpu.sync_copy(outv, out.at[:, pl.ds(base, P)])

        if pipelined:
            stage_prep_fire(0, 0)

            def pair(j, c):
                c0 = 2 * j
                stage_prep_fire(c0 + 1, 1)
                wait_combine_store(c0, 0)

                @pl.when(c0 + 2 < nchunks)
                def _fire_next():
                    stage_prep_fire(c0 + 2, 0)

                wait_combine_store(c0 + 1, 1)
                return c

            lax.fori_loop(0, nchunks // 2, pair, 0)
        else:
            def chunk(kk, c):
                stage_prep_fire(kk, 0)
                wait_combine_store(kk, 0)
                return c

            lax.fori_loop(0, nchunks, chunk, 0)

    return pl.kernel(body, out_type=out_type, mesh=mesh,
                     scratch_types=scratch, interpret=interpret,
                     compiler_params=pltpu.CompilerParams(
                         needs_layout_passes=False,
                         use_tc_tiling_on_sc=False))


def kernel(x, feature_grid_3d, plane0, plane1, plane2):
    B = x.shape[0]
    C = feature_grid_3d.shape[1]
    D, Hg, Wg = feature_grid_3d.shape[2:5]
    assert C == _L and B % (_NW * _L) == 0
    pdims = [(p.shape[2], p.shape[3]) for p in (plane0, plane1, plane2)]
    sizes = [D * Hg * Wg] + [ph * pw for ph, pw in pdims]
    rk = _make_relayout_kernel(C, tuple(sizes))
    flats = rk(feature_grid_3d.reshape(C, -1),
               plane0.reshape(C, -1),
               plane1.reshape(C, -1),
               plane2.reshape(C, -1))
    g3t, p0t, p1t, p2t = (f.reshape(n, 2 * C) for f, n in zip(flats, sizes))
    k = _make_sc_kernel(B, C, (D, Hg, Wg), tuple(pdims))
    return k(x.reshape(-1), g3t, p0t, p1t, p2t)


# speedup vs baseline: 2.3019x; 2.3019x over previous
import jax
import jax.numpy as jnp
from jax import lax
from jax.experimental import pallas as pl
from jax.experimental.pallas import tpu as pltpu
from jax.experimental.pallas import tpu_sc as plsc

_NC, _NS, _L = 2, 16, 16
_NW = _NC * _NS
_P = 128
_PLANE_DIMID = ((0, 1), (0, 2), (1, 2))
_NT = 20


def _make_relayout_kernel(C, sizes, interpret=False):
    U = 1024
    counts = [n // (U * _NW) for n in sizes]
    assert all(n % (U * _NW) == 0 for n in sizes)

    mesh = plsc.VectorSubcoreMesh(core_axis_name="c", subcore_axis_name="s",
                                  num_cores=_NC, num_subcores=_NS)
    out_type = tuple(jax.ShapeDtypeStruct((n * C,), jnp.float32)
                     for n in sizes)
    scratch = [
        pltpu.VMEM((C, U), jnp.float32),
        pltpu.VMEM((C, U), jnp.float32),
        pltpu.VMEM((U * C,), jnp.float32),
        pltpu.SemaphoreType.DMA,
        pltpu.SemaphoreType.DMA,
    ]

    def body(*refs):
        n = len(sizes)
        ins = refs[:n]
        outs = refs[n:2 * n]
        inb0, inb1, stg, sem0, sem1 = refs[2 * n:]
        inbs = (inb0, inb1)
        sems = (sem0, sem1)
        wid = lax.axis_index("s") * _NC + lax.axis_index("c")
        lane = lax.iota(jnp.int32, _L)
        laneC = lane * C

        def do_table(t_in, t_out, nunits):
            ubase = wid * nunits

            def fire_in(i, par):
                off = (ubase + i) * U
                for ch in range(C):
                    pltpu.async_copy(t_in.at[ch, pl.ds(off, U)],
                                     inbs[par].at[ch], sems[par])

            def wait_in(i, par):
                off = (ubase + i) * U
                for ch in range(C):
                    pltpu.make_async_copy(t_in.at[ch, pl.ds(off, U)],
                                          inbs[par].at[ch], sems[par]).wait()

            def process(i, par):
                inb = inbs[par]

                def grp(i16, cc):
                    nidx = laneC + i16 * (_L * C)
                    sl = pl.ds(i16 * _L, _L)
                    for ch in range(C):
                        plsc.store_scatter(stg, [nidx + ch], inb[ch, sl])
                    return cc

                lax.fori_loop(0, U // _L, grp, 0)
                off = (ubase + i) * U
                pltpu.sync_copy(stg, t_out.at[pl.ds(off * C, U * C)])

            fire_in(0, 0)

            def pairj(j, c):
                i0 = 2 * j
                fire_in(i0 + 1, 1)
                wait_in(i0, 0)
                process(i0, 0)

                @pl.when(i0 + 2 < nunits)
                def _next():
                    fire_in(i0 + 2, 0)

                wait_in(i0 + 1, 1)
                process(i0 + 1, 1)
                return c

            assert nunits % 2 == 0
            lax.fori_loop(0, nunits // 2, pairj, 0)

        for t_in, t_out, cnt in zip(ins, outs, counts):
            do_table(t_in, t_out, cnt)

    return pl.kernel(body, out_type=out_type, mesh=mesh,
                     scratch_types=scratch, interpret=interpret,
                     compiler_params=pltpu.CompilerParams(
                         needs_layout_passes=False,
                         use_tc_tiling_on_sc=False))


def _make_sc_kernel(B, C, vol_dims, pdims, interpret=False):
    D, Hg, Wg = vol_dims
    ppt = B // _NW
    P = min(_P, ppt)
    nchunks = ppt // P
    ngroups = P // _L
    pipelined = nchunks % 2 == 0 and nchunks >= 2

    mesh = plsc.VectorSubcoreMesh(core_axis_name="c", subcore_axis_name="s",
                                  num_cores=_NC, num_subcores=_NS)
    out_type = jax.ShapeDtypeStruct((C, B), jnp.float32)
    scratch = [
        pltpu.VMEM((P * 6,), jnp.float32),
        pltpu.VMEM((_NT, P), jnp.int32),
        pltpu.VMEM((_NT, P), jnp.int32),
        pltpu.VMEM((_NT, P), jnp.float32),
        pltpu.VMEM((_NT, P), jnp.float32),
        pltpu.VMEM((_NT, P, C), jnp.float32),
        pltpu.VMEM((_NT, P, C), jnp.float32),
        pltpu.VMEM((C, P), jnp.float32),
        pltpu.SemaphoreType.DMA,
        pltpu.SemaphoreType.DMA,
    ]

    def stream_plan(g3, p0, p1, p2):
        tabs = (g3,) * 8 + (p0,) * 4 + (p1,) * 4 + (p2,) * 4
        return tuple((tab, t) for t, tab in enumerate(tabs))

    def body(xf, g3, p0, p1, p2, out, xv, idx0, idx1, w0, w1, rows0, rows1,
             outv, sem0, sem1):
        wid = lax.axis_index("s") * _NC + lax.axis_index("c")
        tile_base = wid * ppt
        lane = lax.iota(jnp.int32, _L)
        plan = stream_plan(g3, p0, p1, p2)
        idxs = (idx0, idx1)
        rows = (rows0, rows1)
        sems = (sem0, sem1)

        def make_prep(idx_r, w_r):
            def prep_group(g, c):
                sl = pl.ds(g * _L, _L)
                p6 = (lane + g * _L) * 6
                gx = plsc.load_gather(xv, [p6])
                gy = plsc.load_gather(xv, [p6 + 1])
                gz = plsc.load_gather(xv, [p6 + 2])
                coords = (gx, gy, gz)
                ix = (gx + 1.0) * 0.5 * (Wg - 1)
                iy = (gy + 1.0) * 0.5 * (Hg - 1)
                iz = (gz + 1.0) * 0.5 * (D - 1)
                xi = jnp.clip(ix.astype(jnp.int32), 0, Wg - 1)
                yi = jnp.clip(iy.astype(jnp.int32), 0, Hg - 1)
                zi = jnp.clip(iz.astype(jnp.int32), 0, D - 1)
                fx = ix - xi.astype(jnp.float32)
                fy = iy - yi.astype(jnp.float32)
                fz = iz - zi.astype(jnp.float32)
                dx = jnp.minimum(xi + 1, Wg - 1) - xi
                dy = (jnp.minimum(yi + 1, Hg - 1) - yi) * Wg
                dz = (jnp.minimum(zi + 1, D - 1) - zi) * (Wg * Hg)
                base3 = (zi * Hg + yi) * Wg + xi
                idx_r[0, sl] = base3
                idx_r[1, sl] = base3 + dx
                idx_r[2, sl] = base3 + dy
                idx_r[3, sl] = base3 + dy + dx
                idx_r[4, sl] = base3 + dz
                idx_r[5, sl] = base3 + dz + dx
                idx_r[6, sl] = base3 + dz + dy
                idx_r[7, sl] = base3 + dz + dy + dx
                ox = 1.0 - fx
                oy = 1.0 - fy
                oz = 1.0 - fz
                w_r[0, sl] = ox * oy * oz
                w_r[1, sl] = fx * oy * oz
                w_r[2, sl] = ox * fy * oz
                w_r[3, sl] = fx * fy * oz
                w_r[4, sl] = ox * oy * fz
                w_r[5, sl] = fx * oy * fz
                w_r[6, sl] = ox * fy * fz
                w_r[7, sl] = fx * fy * fz
                for k, (a, b) in enumerate(_PLANE_DIMID):
                    PHk, PWk = pdims[k]
                    gu = coords[a]
                    gv = coords[b]
                    iu = (gu + 1.0) * 0.5 * (PWk - 1)
                    iv = (gv + 1.0) * 0.5 * (PHk - 1)
                    ui = jnp.clip(iu.astype(jnp.int32), 0, PWk - 1)
                    vi = jnp.clip(iv.astype(jnp.int32), 0, PHk - 1)
                    fu = iu - ui.astype(jnp.float32)
                    fv = iv - vi.astype(jnp.float32)
                    du = jnp.minimum(ui + 1, PWk - 1) - ui
                    dv = (jnp.minimum(vi + 1, PHk - 1) - vi) * PWk
                    r = vi * PWk + ui
                    t0 = 8 + 4 * k
                    idx_r[t0 + 0, sl] = r
                    idx_r[t0 + 1, sl] = r + du
                    idx_r[t0 + 2, sl] = r + dv
                    idx_r[t0 + 3, sl] = r + dv + du
                    ou = 1.0 - fu
                    ov = 1.0 - fv
                    w_r[t0 + 0, sl] = ou * ov
                    w_r[t0 + 1, sl] = fu * ov
                    w_r[t0 + 2, sl] = ou * fv
                    w_r[t0 + 3, sl] = fu * fv
                return c
            return prep_group

        def make_combine(rows_r, w_r):
            def combine_group(g, c):
                sl = pl.ds(g * _L, _L)
                pvec = lane + g * _L
                wvs = [w_r[t, sl] for t in range(_NT)]
                for f in range(C):
                    fvec = jnp.full((_L,), f, jnp.int32)
                    a = None
                    for t in range(8):
                        tv = jnp.full((_L,), t, jnp.int32)
                        v = plsc.load_gather(rows_r, [tv, pvec, fvec])
                        a = wvs[t] * v if a is None else a + wvs[t] * v
                    for k in range(3):
                        m = None
                        for t in range(8 + 4 * k, 12 + 4 * k):
                            tv = jnp.full((_L,), t, jnp.int32)
                            v = plsc.load_gather(rows_r, [tv, pvec, fvec])
                            m = wvs[t] * v if m is None else m + wvs[t] * v
                        a = a * m
                    outv[f, sl] = a
                return c
            return combine_group

        preps = (make_prep(idx0, w0), make_prep(idx1, w1))
        combines = (make_combine(rows0, w0), make_combine(rows1, w1))

        def stage_prep_fire(cc, par):
            base = tile_base + cc * P
            pltpu.sync_copy(xf.at[pl.ds(base * 6, P * 6)], xv)
            lax.fori_loop(0, ngroups, preps[par], 0)
            for tab, t0 in plan:
                pltpu.async_copy(tab.at[idxs[par].at[t0]],
                                 rows[par].at[t0], sems[par])

        def wait_combine_store(cc, par):
            for tab, t0 in plan:
                pltpu.make_async_copy(tab.at[idxs[par].at[t0]],
                                      rows[par].at[t0], sems[par]).wait()
            lax.fori_loop(0, ngroups, combines[par], 0)
            base = tile_base + cc * P
            pltpu.sync_copy(outv, out.at[:, pl.ds(base, P)])

        if pipelined:
            stage_prep_fire(0, 0)

            def pair(j, c):
                c0 = 2 * j
                stage_prep_fire(c0 + 1, 1)
                wait_combine_store(c0, 0)

                @pl.when(c0 + 2 < nchunks)
                def _fire_next():
                    stage_prep_fire(c0 + 2, 0)

                wait_combine_store(c0 + 1, 1)
                return c

            lax.fori_loop(0, nchunks // 2, pair, 0)
        else:
            def chunk(kk, c):
                stage_prep_fire(kk, 0)
                wait_combine_store(kk, 0)
                return c

            lax.fori_loop(0, nchunks, chunk, 0)

    return pl.kernel(body, out_type=out_type, mesh=mesh,
                     scratch_types=scratch, interpret=interpret,
                     compiler_params=pltpu.CompilerParams(
                         needs_layout_passes=False,
                         use_tc_tiling_on_sc=False))


def kernel(x, feature_grid_3d, plane0, plane1, plane2):
    B = x.shape[0]
    C = feature_grid_3d.shape[1]
    D, Hg, Wg = feature_grid_3d.shape[2:5]
    assert C == _L and B % (_NW * _L) == 0
    pdims = [(p.shape[2], p.shape[3]) for p in (plane0, plane1, plane2)]
    sizes = [D * Hg * Wg] + [ph * pw for ph, pw in pdims]
    rk = _make_relayout_kernel(C, tuple(sizes))
    flats = rk(feature_grid_3d.reshape(C, -1),
               plane0.reshape(C, -1),
               plane1.reshape(C, -1),
               plane2.reshape(C, -1))
    g3t, p0t, p1t, p2t = (f.reshape(n, C) for f, n in zip(flats, sizes))
    k = _make_sc_kernel(B, C, (D, Hg, Wg), tuple(pdims))
    return k(x.reshape(-1), g3t, p0t, p1t, p2t)


# speedup vs baseline: 2.9514x; 1.2822x over previous
import jax
import jax.numpy as jnp
from jax import lax
from jax.experimental import pallas as pl
from jax.experimental.pallas import tpu as pltpu
from jax.experimental.pallas import tpu_sc as plsc

_NC, _NS, _L = 2, 16, 16
_NW = _NC * _NS
_P = 128
_PLANE_DIMID = ((0, 1), (0, 2), (1, 2))
_NT = 10


def _bf16_word(lo_f32, hi_f32):
    lo = plsc.bitcast(lo_f32, jnp.uint32)
    hi = plsc.bitcast(hi_f32, jnp.uint32)
    lo = jnp.right_shift(lo + jnp.uint32(0x8000), jnp.uint32(16))
    hi = (hi + jnp.uint32(0x8000)) & jnp.uint32(0xFFFF0000)
    return plsc.bitcast(lo | hi, jnp.int32)


def _make_relayout_kernel(C, sizes, interpret=False):
    U = 1024
    E = _L
    counts = [n // (U * _NW) for n in sizes]
    assert all(n % (U * _NW) == 0 for n in sizes)

    mesh = plsc.VectorSubcoreMesh(core_axis_name="c", subcore_axis_name="s",
                                  num_cores=_NC, num_subcores=_NS)
    out_type = tuple(jax.ShapeDtypeStruct((n * C,), jnp.int32)
                     for n in sizes)
    scratch = [
        pltpu.VMEM((C, U + E), jnp.float32),
        pltpu.VMEM((C, U + E), jnp.float32),
        pltpu.VMEM((U * C,), jnp.int32),
        pltpu.SemaphoreType.DMA,
        pltpu.SemaphoreType.DMA,
    ]

    def body(*refs):
        n = len(sizes)
        ins = refs[:n]
        outs = refs[n:2 * n]
        inb0, inb1, stg, sem0, sem1 = refs[2 * n:]
        inbs = (inb0, inb1)
        sems = (sem0, sem1)
        wid = lax.axis_index("s") * _NC + lax.axis_index("c")
        lane = lax.iota(jnp.int32, _L)
        laneC = lane * C

        def do_table(t_in, t_out, nunits, ntot):
            ubase = wid * nunits

            def in_descs(i, par, make):
                off = (ubase + i) * U
                full = off + U + E <= ntot

                @pl.when(full)
                def _full():
                    for ch in range(C):
                        make(t_in.at[ch, pl.ds(off, U + E)],
                             inbs[par].at[ch], sems[par])

                @pl.when(jnp.logical_not(full))
                def _short():
                    for ch in range(C):
                        make(t_in.at[ch, pl.ds(off, U)],
                             inbs[par].at[ch, pl.ds(0, U)], sems[par])

            def fire_in(i, par):
                in_descs(i, par, pltpu.async_copy)

            def wait_in(i, par):
                in_descs(i, par,
                         lambda s, d, m: pltpu.make_async_copy(s, d, m).wait())

            def process(i, par):
                inb = inbs[par]

                def grp(i16, cc):
                    nidx = laneC + i16 * (_L * C)
                    sl0 = pl.ds(i16 * _L, _L)
                    for w in range(C // 2):
                        lo = inb[2 * w, sl0]
                        hi = inb[2 * w + 1, sl0]
                        word = _bf16_word(lo, hi)
                        plsc.store_scatter(stg, [nidx + w], word)
                        lo1 = plsc.load_gather(
                            inb, [jnp.full((_L,), 2 * w, jnp.int32),
                                  lane + (i16 * _L + 1)])
                        hi1 = plsc.load_gather(
                            inb, [jnp.full((_L,), 2 * w + 1, jnp.int32),
                                  lane + (i16 * _L + 1)])
                        word1 = _bf16_word(lo1, hi1)
                        plsc.store_scatter(stg, [nidx + (C // 2 + w)], word1)
                    return cc

                lax.fori_loop(0, U // _L, grp, 0)
                off = (ubase + i) * U
                pltpu.sync_copy(stg, t_out.at[pl.ds(off * C, U * C)])

            fire_in(0, 0)

            def pairj(j, c):
                i0 = 2 * j
                fire_in(i0 + 1, 1)
                wait_in(i0, 0)
                process(i0, 0)

                @pl.when(i0 + 2 < nunits)
                def _next():
                    fire_in(i0 + 2, 0)

                wait_in(i0 + 1, 1)
                process(i0 + 1, 1)
                return c

            assert nunits % 2 == 0
            lax.fori_loop(0, nunits // 2, pairj, 0)

        for t_in, t_out, cnt, ntot in zip(ins, outs, counts, sizes):
            do_table(t_in, t_out, cnt, ntot)

    return pl.kernel(body, out_type=out_type, mesh=mesh,
                     scratch_types=scratch, interpret=interpret,
                     compiler_params=pltpu.CompilerParams(
                         needs_layout_passes=False,
                         use_tc_tiling_on_sc=False))


def _make_sc_kernel(B, C, vol_dims, pdims, interpret=False):
    D, Hg, Wg = vol_dims
    ppt = B // _NW
    P = min(_P, ppt)
    nchunks = ppt // P
    ngroups = P // _L
    pipelined = nchunks % 2 == 0 and nchunks >= 2

    mesh = plsc.VectorSubcoreMesh(core_axis_name="c", subcore_axis_name="s",
                                  num_cores=_NC, num_subcores=_NS)
    out_type = jax.ShapeDtypeStruct((C, B), jnp.float32)
    scratch = [
        pltpu.VMEM((P * 6,), jnp.float32),
        pltpu.VMEM((_NT, P), jnp.int32),
        pltpu.VMEM((_NT, P), jnp.int32),
        pltpu.VMEM((2 * _NT, P), jnp.float32),
        pltpu.VMEM((2 * _NT, P), jnp.float32),
        pltpu.VMEM((_NT, P, C), jnp.int32),
        pltpu.VMEM((_NT, P, C), jnp.int32),
        pltpu.VMEM((C, P), jnp.float32),
        pltpu.SemaphoreType.DMA,
        pltpu.SemaphoreType.DMA,
    ]

    def body(xf, g3, p0, p1, p2, out, xv, idx0, idx1, w0, w1, rows0, rows1,
             outv, sem0, sem1):
        wid = lax.axis_index("s") * _NC + lax.axis_index("c")
        tile_base = wid * ppt
        lane = lax.iota(jnp.int32, _L)
        tabs = (g3,) * 4 + (p0,) * 2 + (p1,) * 2 + (p2,) * 2
        idxs = (idx0, idx1)
        rows = (rows0, rows1)
        sems = (sem0, sem1)

        def make_prep(idx_r, w_r):
            def prep_group(g, c):
                sl = pl.ds(g * _L, _L)
                p6 = (lane + g * _L) * 6
                gx = plsc.load_gather(xv, [p6])
                gy = plsc.load_gather(xv, [p6 + 1])
                gz = plsc.load_gather(xv, [p6 + 2])
                coords = (gx, gy, gz)
                ix = (gx + 1.0) * 0.5 * (Wg - 1)
                iy = (gy + 1.0) * 0.5 * (Hg - 1)
                iz = (gz + 1.0) * 0.5 * (D - 1)
                xi = jnp.clip(ix.astype(jnp.int32), 0, Wg - 1)
                yi = jnp.clip(iy.astype(jnp.int32), 0, Hg - 1)
                zi = jnp.clip(iz.astype(jnp.int32), 0, D - 1)
                fx = ix - xi.astype(jnp.float32)
                fy = iy - yi.astype(jnp.float32)
                fz = iz - zi.astype(jnp.float32)
                dy = (jnp.minimum(yi + 1, Hg - 1) - yi) * Wg
                dz = (jnp.minimum(zi + 1, D - 1) - zi) * (Wg * Hg)
                base3 = (zi * Hg + yi) * Wg + xi
                idx_r[0, sl] = base3
                idx_r[1, sl] = base3 + dy
                idx_r[2, sl] = base3 + dz
                idx_r[3, sl] = base3 + dz + dy
                ox = 1.0 - fx
                oy = 1.0 - fy
                oz = 1.0 - fz
                wzy = (oz * oy, oz * fy, fz * oy, fz * fy)
                for r in range(4):
                    w_r[2 * r, sl] = wzy[r] * ox
                    w_r[2 * r + 1, sl] = wzy[r] * fx
                for k, (a, b) in enumerate(_PLANE_DIMID):
                    PHk, PWk = pdims[k]
                    gu = coords[a]
                    gv = coords[b]
                    iu = (gu + 1.0) * 0.5 * (PWk - 1)
                    iv = (gv + 1.0) * 0.5 * (PHk - 1)
                    ui = jnp.clip(iu.astype(jnp.int32), 0, PWk - 1)
                    vi = jnp.clip(iv.astype(jnp.int32), 0, PHk - 1)
                    fu = iu - ui.astype(jnp.float32)
                    fv = iv - vi.astype(jnp.float32)
                    dv = (jnp.minimum(vi + 1, PHk - 1) - vi) * PWk
                    r0 = vi * PWk + ui
                    t0 = 4 + 2 * k
                    idx_r[t0, sl] = r0
                    idx_r[t0 + 1, sl] = r0 + dv
                    ou = 1.0 - fu
                    ov = 1.0 - fv
                    w_r[8 + 4 * k + 0, sl] = ov * ou
                    w_r[8 + 4 * k + 1, sl] = ov * fu
                    w_r[8 + 4 * k + 2, sl] = fv * ou
                    w_r[8 + 4 * k + 3, sl] = fv * fu
                return c
            return prep_group

        def unpack2(word):
            u = plsc.bitcast(word, jnp.uint32)
            flo = plsc.bitcast(jnp.left_shift(u, jnp.uint32(16)), jnp.float32)
            fhi = plsc.bitcast(u & jnp.uint32(0xFFFF0000), jnp.float32)
            return flo, fhi

        def make_combine(rows_r, w_r):
            def combine_group(g, c):
                sl = pl.ds(g * _L, _L)
                pvec = lane + g * _L
                wvs = [w_r[t, sl] for t in range(2 * _NT)]
                H = C // 2
                for w in range(H):
                    wv0 = jnp.full((_L,), w, jnp.int32)
                    wv1 = jnp.full((_L,), w + H, jnp.int32)
                    alo = None
                    ahi = None
                    for t in range(4):
                        tv = jnp.full((_L,), t, jnp.int32)
                        x0 = plsc.load_gather(rows_r, [tv, pvec, wv0])
                        x1 = plsc.load_gather(rows_r, [tv, pvec, wv1])
                        l0, h0 = unpack2(x0)
                        l1, h1 = unpack2(x1)
                        ul = wvs[2 * t] * l0 + wvs[2 * t + 1] * l1
                        uh = wvs[2 * t] * h0 + wvs[2 * t + 1] * h1
                        alo = ul if alo is None else alo + ul
                        ahi = uh if ahi is None else ahi + uh
                    for k in range(3):
                        mlo = None
                        mhi = None
                        for h in range(2):
                            t = 4 + 2 * k + h
                            tv = jnp.full((_L,), t, jnp.int32)
                            x0 = plsc.load_gather(rows_r, [tv, pvec, wv0])
                            x1 = plsc.load_gather(rows_r, [tv, pvec, wv1])
                            l0, h0 = unpack2(x0)
                            l1, h1 = unpack2(x1)
                            wa = wvs[8 + 4 * k + 2 * h]
                            wb = wvs[8 + 4 * k + 2 * h + 1]
                            ul = wa * l0 + wb * l1
                            uh = wa * h0 + wb * h1
                            mlo = ul if mlo is None else mlo + ul
                            mhi = uh if mhi is None else mhi + uh
                        alo = alo * mlo
                        ahi = ahi * mhi
                    outv[2 * w, sl] = alo
                    outv[2 * w + 1, sl] = ahi
                return c
            return combine_group

        preps = (make_prep(idx0, w0), make_prep(idx1, w1))
        combines = (make_combine(rows0, w0), make_combine(rows1, w1))

        def stage_prep_fire(cc, par):
            base = tile_base + cc * P
            pltpu.sync_copy(xf.at[pl.ds(base * 6, P * 6)], xv)
            lax.fori_loop(0, ngroups, preps[par], 0)
            for t in range(_NT):
                pltpu.async_copy(tabs[t].at[idxs[par].at[t]], rows[par].at[t],
                                 sems[par])

        def wait_combine_store(cc, par):
            for t in range(_NT):
                pltpu.make_async_copy(tabs[t].at[idxs[par].at[t]],
                                      rows[par].at[t], sems[par]).wait()
            lax.fori_loop(0, ngroups, combines[par], 0)
            base = tile_base + cc * P
            pltpu.sync_copy(outv, out.at[:, pl.ds(base, P)])

        if pipelined:
            stage_prep_fire(0, 0)

            def pair(j, c):
                c0 = 2 * j
                stage_prep_fire(c0 + 1, 1)
                wait_combine_store(c0, 0)

                @pl.when(c0 + 2 < nchunks)
                def _fire_next():
                    stage_prep_fire(c0 + 2, 0)

                wait_combine_store(c0 + 1, 1)
                return c

            lax.fori_loop(0, nchunks // 2, pair, 0)
        else:
            def chunk(kk, c):
                stage_prep_fire(kk, 0)
                wait_combine_store(kk, 0)
                return c

            lax.fori_loop(0, nchunks, chunk, 0)

    return pl.kernel(body, out_type=out_type, mesh=mesh,
                     scratch_types=scratch, interpret=interpret,
                     compiler_params=pltpu.CompilerParams(
                         needs_layout_passes=False,
                         use_tc_tiling_on_sc=False))


def kernel(x, feature_grid_3d, plane0, plane1, plane2):
    B = x.shape[0]
    C = feature_grid_3d.shape[1]
    D, Hg, Wg = feature_grid_3d.shape[2:5]
    assert C == _L and B % (_NW * _L) == 0
    pdims = [(p.shape[2], p.shape[3]) for p in (plane0, plane1, plane2)]
    sizes = [D * Hg * Wg] + [ph * pw for ph, pw in pdims]
    rk = _make_relayout_kernel(C, tuple(sizes))
    flats = rk(feature_grid_3d.reshape(C, -1),
               plane0.reshape(C, -1),
               plane1.reshape(C, -1),
               plane2.reshape(C, -1))
    g3t, p0t, p1t, p2t = (f.reshape(n, C) for f, n in zip(flats, sizes))
    k = _make_sc_kernel(B, C, (D, Hg, Wg), tuple(pdims))
    return k(x.reshape(-1), g3t, p0t, p1t, p2t)


# speedup vs baseline: 3.8324x; 1.2985x over previous
import jax
import jax.numpy as jnp
from jax import lax
from jax.experimental import pallas as pl
from jax.experimental.pallas import tpu as pltpu
from jax.experimental.pallas import tpu_sc as plsc

_NC, _NS, _L = 2, 16, 16
_NW = _NC * _NS
_P = 128
_PLANE_DIMID = ((0, 1), (0, 2), (1, 2))
_NT = 10


def _bf16_word(lo_f32, hi_f32):
    lo = plsc.bitcast(lo_f32, jnp.uint32)
    hi = plsc.bitcast(hi_f32, jnp.uint32)
    lo = jnp.right_shift(lo + jnp.uint32(0x8000), jnp.uint32(16))
    hi = (hi + jnp.uint32(0x8000)) & jnp.uint32(0xFFFF0000)
    return plsc.bitcast(lo | hi, jnp.int32)


def _make_relayout_kernel(C, sizes, interpret=False):
    U = 1024
    E = _L
    counts = [n // (U * _NW) for n in sizes]
    assert all(n % (U * _NW) == 0 for n in sizes)

    mesh = plsc.VectorSubcoreMesh(core_axis_name="c", subcore_axis_name="s",
                                  num_cores=_NC, num_subcores=_NS)
    out_type = tuple(jax.ShapeDtypeStruct((n * C,), jnp.int32)
                     for n in sizes)
    scratch = [
        pltpu.VMEM((C, U + E), jnp.float32),
        pltpu.VMEM((C, U + E), jnp.float32),
        pltpu.VMEM((U * C,), jnp.int32),
        pltpu.SemaphoreType.DMA,
        pltpu.SemaphoreType.DMA,
    ]

    def body(*refs):
        n = len(sizes)
        ins = refs[:n]
        outs = refs[n:2 * n]
        inb0, inb1, stg, sem0, sem1 = refs[2 * n:]
        inbs = (inb0, inb1)
        sems = (sem0, sem1)
        wid = lax.axis_index("s") * _NC + lax.axis_index("c")
        lane = lax.iota(jnp.int32, _L)
        laneC = lane * C

        def do_table(t_in, t_out, nunits, ntot):
            ubase = wid * nunits

            def in_descs(i, par, make):
                off = (ubase + i) * U
                full = off + U + E <= ntot

                @pl.when(full)
                def _full():
                    for ch in range(C):
                        make(t_in.at[ch, pl.ds(off, U + E)],
                             inbs[par].at[ch], sems[par])

                @pl.when(jnp.logical_not(full))
                def _short():
                    for ch in range(C):
                        make(t_in.at[ch, pl.ds(off, U)],
                             inbs[par].at[ch, pl.ds(0, U)], sems[par])

            def fire_in(i, par):
                in_descs(i, par, pltpu.async_copy)

            def wait_in(i, par):
                in_descs(i, par,
                         lambda s, d, m: pltpu.make_async_copy(s, d, m).wait())

            def process(i, par):
                inb = inbs[par]

                def grp(i16, cc):
                    nidx = laneC + i16 * (_L * C)
                    sl0 = pl.ds(i16 * _L, _L)
                    edge = (lane + i16 * _L) >= 1
                    for w in range(C // 2):
                        lo = inb[2 * w, sl0]
                        hi = inb[2 * w + 1, sl0]
                        word = _bf16_word(lo, hi)
                        plsc.store_scatter(stg, [nidx + w], word)
                        plsc.store_scatter(
                            stg, [jnp.maximum(nidx + (w - C // 2), 0)], word,
                            mask=edge)
                    return cc

                lax.fori_loop(0, U // _L, grp, 0)
                epi = lane == 0
                for w in range(C // 2):
                    lo = inb[2 * w, pl.ds(U, _L)]
                    hi = inb[2 * w + 1, pl.ds(U, _L)]
                    word = _bf16_word(lo, hi)
                    tgt = jnp.full((_L,), (U - 1) * C + C // 2 + w, jnp.int32)
                    plsc.store_scatter(stg, [tgt], word, mask=epi)
                off = (ubase + i) * U
                pltpu.sync_copy(stg, t_out.at[pl.ds(off * C, U * C)])

            fire_in(0, 0)

            def pairj(j, c):
                i0 = 2 * j
                fire_in(i0 + 1, 1)
                wait_in(i0, 0)
                process(i0, 0)

                @pl.when(i0 + 2 < nunits)
                def _next():
                    fire_in(i0 + 2, 0)

                wait_in(i0 + 1, 1)
                process(i0 + 1, 1)
                return c

            assert nunits % 2 == 0
            lax.fori_loop(0, nunits // 2, pairj, 0)

        for t_in, t_out, cnt, ntot in zip(ins, outs, counts, sizes):
            do_table(t_in, t_out, cnt, ntot)

    return pl.kernel(body, out_type=out_type, mesh=mesh,
                     scratch_types=scratch, interpret=interpret,
                     compiler_params=pltpu.CompilerParams(
                         needs_layout_passes=False,
                         use_tc_tiling_on_sc=False))


def _make_sc_kernel(B, C, vol_dims, pdims, interpret=False):
    D, Hg, Wg = vol_dims
    ppt = B // _NW
    P = min(_P, ppt)
    nchunks = ppt // P
    ngroups = P // _L
    pipelined = nchunks % 2 == 0 and nchunks >= 2

    mesh = plsc.VectorSubcoreMesh(core_axis_name="c", subcore_axis_name="s",
                                  num_cores=_NC, num_subcores=_NS)
    out_type = jax.ShapeDtypeStruct((C, B), jnp.float32)
    scratch = [
        pltpu.VMEM((P * 6,), jnp.float32),
        pltpu.VMEM((_NT, P), jnp.int32),
        pltpu.VMEM((_NT, P), jnp.int32),
        pltpu.VMEM((2 * _NT, P), jnp.float32),
        pltpu.VMEM((2 * _NT, P), jnp.float32),
        pltpu.VMEM((_NT, P, C), jnp.int32),
        pltpu.VMEM((_NT, P, C), jnp.int32),
        pltpu.VMEM((C, P), jnp.float32),
        pltpu.SemaphoreType.DMA,
        pltpu.SemaphoreType.DMA,
    ]

    def body(xf, g3, p0, p1, p2, out, xv, idx0, idx1, w0, w1, rows0, rows1,
             outv, sem0, sem1):
        wid = lax.axis_index("s") * _NC + lax.axis_index("c")
        tile_base = wid * ppt
        lane = lax.iota(jnp.int32, _L)
        tabs = (g3,) * 4 + (p0,) * 2 + (p1,) * 2 + (p2,) * 2
        idxs = (idx0, idx1)
        rows = (rows0, rows1)
        sems = (sem0, sem1)

        def make_prep(idx_r, w_r):
            def prep_group(g, c):
                sl = pl.ds(g * _L, _L)
                p6 = (lane + g * _L) * 6
                gx = plsc.load_gather(xv, [p6])
                gy = plsc.load_gather(xv, [p6 + 1])
                gz = plsc.load_gather(xv, [p6 + 2])
                coords = (gx, gy, gz)
                ix = (gx + 1.0) * 0.5 * (Wg - 1)
                iy = (gy + 1.0) * 0.5 * (Hg - 1)
                iz = (gz + 1.0) * 0.5 * (D - 1)
                xi = jnp.clip(ix.astype(jnp.int32), 0, Wg - 1)
                yi = jnp.clip(iy.astype(jnp.int32), 0, Hg - 1)
                zi = jnp.clip(iz.astype(jnp.int32), 0, D - 1)
                fx = ix - xi.astype(jnp.float32)
                fy = iy - yi.astype(jnp.float32)
                fz = iz - zi.astype(jnp.float32)
                dy = (jnp.minimum(yi + 1, Hg - 1) - yi) * Wg
                dz = (jnp.minimum(zi + 1, D - 1) - zi) * (Wg * Hg)
                base3 = (zi * Hg + yi) * Wg + xi
                idx_r[0, sl] = base3
                idx_r[1, sl] = base3 + dy
                idx_r[2, sl] = base3 + dz
                idx_r[3, sl] = base3 + dz + dy
                ox = 1.0 - fx
                oy = 1.0 - fy
                oz = 1.0 - fz
                wzy = (oz * oy, oz * fy, fz * oy, fz * fy)
                for r in range(4):
                    w_r[2 * r, sl] = wzy[r] * ox
                    w_r[2 * r + 1, sl] = wzy[r] * fx
                for k, (a, b) in enumerate(_PLANE_DIMID):
                    PHk, PWk = pdims[k]
                    gu = coords[a]
                    gv = coords[b]
                    iu = (gu + 1.0) * 0.5 * (PWk - 1)
                    iv = (gv + 1.0) * 0.5 * (PHk - 1)
                    ui = jnp.clip(iu.astype(jnp.int32), 0, PWk - 1)
                    vi = jnp.clip(iv.astype(jnp.int32), 0, PHk - 1)
                    fu = iu - ui.astype(jnp.float32)
                    fv = iv - vi.astype(jnp.float32)
                    dv = (jnp.minimum(vi + 1, PHk - 1) - vi) * PWk
                    r0 = vi * PWk + ui
                    t0 = 4 + 2 * k
                    idx_r[t0, sl] = r0
                    idx_r[t0 + 1, sl] = r0 + dv
                    ou = 1.0 - fu
                    ov = 1.0 - fv
                    w_r[8 + 4 * k + 0, sl] = ov * ou
                    w_r[8 + 4 * k + 1, sl] = ov * fu
                    w_r[8 + 4 * k + 2, sl] = fv * ou
                    w_r[8 + 4 * k + 3, sl] = fv * fu
                return c
            return prep_group

        def unpack2(word):
            u = plsc.bitcast(word, jnp.uint32)
            flo = plsc.bitcast(jnp.left_shift(u, jnp.uint32(16)), jnp.float32)
            fhi = plsc.bitcast(u & jnp.uint32(0xFFFF0000), jnp.float32)
            return flo, fhi

        def make_combine(rows_r, w_r):
            def combine_group(g, c):
                sl = pl.ds(g * _L, _L)
                pvec = lane + g * _L
                wvs = [w_r[t, sl] for t in range(2 * _NT)]
                H = C // 2
                for w in range(H):
                    wv0 = jnp.full((_L,), w, jnp.int32)
                    wv1 = jnp.full((_L,), w + H, jnp.int32)
                    alo = None
                    ahi = None
                    for t in range(4):
                        tv = jnp.full((_L,), t, jnp.int32)
                        x0 = plsc.load_gather(rows_r, [tv, pvec, wv0])
                        x1 = plsc.load_gather(rows_r, [tv, pvec, wv1])
                        l0, h0 = unpack2(x0)
                        l1, h1 = unpack2(x1)
                        ul = wvs[2 * t] * l0 + wvs[2 * t + 1] * l1
                        uh = wvs[2 * t] * h0 + wvs[2 * t + 1] * h1
                        alo = ul if alo is None else alo + ul
                        ahi = uh if ahi is None else ahi + uh
                    for k in range(3):
                        mlo = None
                        mhi = None
                        for h in range(2):
                            t = 4 + 2 * k + h
                            tv = jnp.full((_L,), t, jnp.int32)
                            x0 = plsc.load_gather(rows_r, [tv, pvec, wv0])
                            x1 = plsc.load_gather(rows_r, [tv, pvec, wv1])
                            l0, h0 = unpack2(x0)
                            l1, h1 = unpack2(x1)
                            wa = wvs[8 + 4 * k + 2 * h]
                            wb = wvs[8 + 4 * k + 2 * h + 1]
                            ul = wa * l0 + wb * l1
                            uh = wa * h0 + wb * h1
                            mlo = ul if mlo is None else mlo + ul
                            mhi = uh if mhi is None else mhi + uh
                        alo = alo * mlo
                        ahi = ahi * mhi
                    outv[2 * w, sl] = alo
                    outv[2 * w + 1, sl] = ahi
                return c
            return combine_group

        preps = (make_prep(idx0, w0), make_prep(idx1, w1))
        combines = (make_combine(rows0, w0), make_combine(rows1, w1))

        def stage_prep_fire(cc, par):
            base = tile_base + cc * P
            pltpu.sync_copy(xf.at[pl.ds(base * 6, P * 6)], xv)
            lax.fori_loop(0, ngroups, preps[par], 0)
            for t in range(_NT):
                pltpu.async_copy(tabs[t].at[idxs[par].at[t]], rows[par].at[t],
                                 sems[par])

        def wait_combine_store(cc, par):
            for t in range(_NT):
                pltpu.make_async_copy(tabs[t].at[idxs[par].at[t]],
                                      rows[par].at[t], sems[par]).wait()
            lax.fori_loop(0, ngroups, combines[par], 0)
            base = tile_base + cc * P
            pltpu.sync_copy(outv, out.at[:, pl.ds(base, P)])

        if pipelined:
            stage_prep_fire(0, 0)

            def pair(j, c):
                c0 = 2 * j
                stage_prep_fire(c0 + 1, 1)
                wait_combine_store(c0, 0)

                @pl.when(c0 + 2 < nchunks)
                def _fire_next():
                    stage_prep_fire(c0 + 2, 0)

                wait_combine_store(c0 + 1, 1)
                return c

            lax.fori_loop(0, nchunks // 2, pair, 0)
        else:
            def chunk(kk, c):
                stage_prep_fire(kk, 0)
                wait_combine_store(kk, 0)
                return c

            lax.fori_loop(0, nchunks, chunk, 0)

    return pl.kernel(body, out_type=out_type, mesh=mesh,
                     scratch_types=scratch, interpret=interpret,
                     compiler_params=pltpu.CompilerParams(
                         needs_layout_passes=False,
                         use_tc_tiling_on_sc=False))


def kernel(x, feature_grid_3d, plane0, plane1, plane2):
    B = x.shape[0]
    C = feature_grid_3d.shape[1]
    D, Hg, Wg = feature_grid_3d.shape[2:5]
    assert C == _L and B % (_NW * _L) == 0
    pdims = [(p.shape[2], p.shape[3]) for p in (plane0, plane1, plane2)]
    sizes = [D * Hg * Wg] + [ph * pw for ph, pw in pdims]
    rk = _make_relayout_kernel(C, tuple(sizes))
    flats = rk(feature_grid_3d.reshape(C, -1),
               plane0.reshape(C, -1),
               plane1.reshape(C, -1),
               plane2.reshape(C, -1))
    g3t, p0t, p1t, p2t = (f.reshape(n, C) for f, n in zip(flats, sizes))
    k = _make_sc_kernel(B, C, (D, Hg, Wg), tuple(pdims))
    return k(x.reshape(-1), g3t, p0t, p1t, p2t)


# speedup vs baseline: 4.0904x; 1.0673x over previous
import jax
import jax.numpy as jnp
from jax import lax
from jax.experimental import pallas as pl
from jax.experimental.pallas import tpu as pltpu
from jax.experimental.pallas import tpu_sc as plsc

_NC, _NS, _L = 2, 16, 16
_NW = _NC * _NS
_P = 128
_PLANE_DIMID = ((0, 1), (0, 2), (1, 2))
_NT = 10


def _bf16_word(lo_f32, hi_f32):
    lo = plsc.bitcast(lo_f32, jnp.uint32)
    hi = plsc.bitcast(hi_f32, jnp.uint32)
    lo = jnp.right_shift(lo + jnp.uint32(0x8000), jnp.uint32(16))
    hi = (hi + jnp.uint32(0x8000)) & jnp.uint32(0xFFFF0000)
    return plsc.bitcast(lo | hi, jnp.int32)


def _make_relayout_kernel(C, sizes, interpret=False):
    U = 1024
    E = _L
    counts = [n // (U * _NW) for n in sizes]
    assert all(n % (U * _NW) == 0 for n in sizes)

    mesh = plsc.VectorSubcoreMesh(core_axis_name="c", subcore_axis_name="s",
                                  num_cores=_NC, num_subcores=_NS)
    out_type = tuple(jax.ShapeDtypeStruct((n * C,), jnp.int32)
                     for n in sizes)
    scratch = [
        pltpu.VMEM((C, U + E), jnp.float32),
        pltpu.VMEM((C, U + E), jnp.float32),
        pltpu.VMEM((U * C,), jnp.int32),
        pltpu.VMEM((U * C,), jnp.int32),
        pltpu.SemaphoreType.DMA,
        pltpu.SemaphoreType.DMA,
        pltpu.SemaphoreType.DMA,
        pltpu.SemaphoreType.DMA,
    ]

    def body(*refs):
        n = len(sizes)
        ins = refs[:n]
        outs = refs[n:2 * n]
        inb0, inb1, stg0, stg1, sem0, sem1, osem0, osem1 = refs[2 * n:]
        inbs = (inb0, inb1)
        stgs = (stg0, stg1)
        sems = (sem0, sem1)
        osems = (osem0, osem1)
        wid = lax.axis_index("s") * _NC + lax.axis_index("c")
        lane = lax.iota(jnp.int32, _L)
        laneC = lane * C

        def do_table(t_in, t_out, nunits, ntot):
            ubase = wid * nunits

            def in_descs(i, par, make):
                off = (ubase + i) * U
                full = off + U + E <= ntot

                @pl.when(full)
                def _full():
                    for ch in range(C):
                        make(t_in.at[ch, pl.ds(off, U + E)],
                             inbs[par].at[ch], sems[par])

                @pl.when(jnp.logical_not(full))
                def _short():
                    for ch in range(C):
                        make(t_in.at[ch, pl.ds(off, U)],
                             inbs[par].at[ch, pl.ds(0, U)], sems[par])

            def fire_in(i, par):
                in_descs(i, par, pltpu.async_copy)

            def wait_in(i, par):
                in_descs(i, par,
                         lambda s, d, m: pltpu.make_async_copy(s, d, m).wait())

            def process(i, par):
                inb = inbs[par]
                stg = stgs[par]
                off = (ubase + i) * U
                dst = t_out.at[pl.ds(off * C, U * C)]

                @pl.when(i >= 2)
                def _wait_prev_out():
                    pltpu.make_async_copy(stg, dst, osems[par]).wait()

                def grp(i16, cc):
                    nidx = laneC + i16 * (_L * C)
                    sl0 = pl.ds(i16 * _L, _L)
                    edge = (lane + i16 * _L) >= 1
                    for w in range(C // 2):
                        lo = inb[2 * w, sl0]
                        hi = inb[2 * w + 1, sl0]
                        word = _bf16_word(lo, hi)
                        plsc.store_scatter(stg, [nidx + w], word)
                        plsc.store_scatter(
                            stg, [jnp.maximum(nidx + (w - C // 2), 0)], word,
                            mask=edge)
                    return cc

                lax.fori_loop(0, U // _L, grp, 0)
                epi = lane == 0
                for w in range(C // 2):
                    lo = inb[2 * w, pl.ds(U, _L)]
                    hi = inb[2 * w + 1, pl.ds(U, _L)]
                    word = _bf16_word(lo, hi)
                    tgt = jnp.full((_L,), (U - 1) * C + C // 2 + w, jnp.int32)
                    plsc.store_scatter(stg, [tgt], word, mask=epi)
                pltpu.async_copy(stg, dst, osems[par])

            fire_in(0, 0)

            def pairj(j, c):
                i0 = 2 * j
                fire_in(i0 + 1, 1)
                wait_in(i0, 0)
                process(i0, 0)

                @pl.when(i0 + 2 < nunits)
                def _next():
                    fire_in(i0 + 2, 0)

                wait_in(i0 + 1, 1)
                process(i0 + 1, 1)
                return c

            assert nunits % 2 == 0
            lax.fori_loop(0, nunits // 2, pairj, 0)
            for par, i in ((0, nunits - 2), (1, nunits - 1)):
                off = (ubase + i) * U
                pltpu.make_async_copy(
                    stgs[par], t_out.at[pl.ds(off * C, U * C)],
                    osems[par]).wait()

        for t_in, t_out, cnt, ntot in zip(ins, outs, counts, sizes):
            do_table(t_in, t_out, cnt, ntot)

    return pl.kernel(body, out_type=out_type, mesh=mesh,
                     scratch_types=scratch, interpret=interpret,
                     compiler_params=pltpu.CompilerParams(
                         needs_layout_passes=False,
                         use_tc_tiling_on_sc=False))


def _make_sc_kernel(B, C, vol_dims, pdims, interpret=False):
    D, Hg, Wg = vol_dims
    ppt = B // _NW
    P = min(_P, ppt)
    nchunks = ppt // P
    ngroups = P // _L
    pipelined = nchunks % 2 == 0 and nchunks >= 2

    mesh = plsc.VectorSubcoreMesh(core_axis_name="c", subcore_axis_name="s",
                                  num_cores=_NC, num_subcores=_NS)
    out_type = jax.ShapeDtypeStruct((C, B), jnp.float32)
    scratch = [
        pltpu.VMEM((P * 6,), jnp.float32),
        pltpu.VMEM((_NT, P), jnp.int32),
        pltpu.VMEM((_NT, P), jnp.int32),
        pltpu.VMEM((2 * _NT, P), jnp.float32),
        pltpu.VMEM((2 * _NT, P), jnp.float32),
        pltpu.VMEM((_NT, P, C), jnp.int32),
        pltpu.VMEM((_NT, P, C), jnp.int32),
        pltpu.VMEM((C, P), jnp.float32),
        pltpu.SemaphoreType.DMA,
        pltpu.SemaphoreType.DMA,
    ]

    def body(xf, g3, p0, p1, p2, out, xv, idx0, idx1, w0, w1, rows0, rows1,
             outv, sem0, sem1):
        wid = lax.axis_index("s") * _NC + lax.axis_index("c")
        tile_base = wid * ppt
        lane = lax.iota(jnp.int32, _L)
        tabs = (g3,) * 4 + (p0,) * 2 + (p1,) * 2 + (p2,) * 2
        idxs = (idx0, idx1)
        rows = (rows0, rows1)
        sems = (sem0, sem1)

        def make_prep(idx_r, w_r):
            def prep_group(g, c):
                sl = pl.ds(g * _L, _L)
                p6 = (lane + g * _L) * 6
                gx = plsc.load_gather(xv, [p6])
                gy = plsc.load_gather(xv, [p6 + 1])
                gz = plsc.load_gather(xv, [p6 + 2])
                coords = (gx, gy, gz)
                ix = (gx + 1.0) * 0.5 * (Wg - 1)
                iy = (gy + 1.0) * 0.5 * (Hg - 1)
                iz = (gz + 1.0) * 0.5 * (D - 1)
                xi = jnp.clip(ix.astype(jnp.int32), 0, Wg - 1)
                yi = jnp.clip(iy.astype(jnp.int32), 0, Hg - 1)
                zi = jnp.clip(iz.astype(jnp.int32), 0, D - 1)
                fx = ix - xi.astype(jnp.float32)
                fy = iy - yi.astype(jnp.float32)
                fz = iz - zi.astype(jnp.float32)
                dy = (jnp.minimum(yi + 1, Hg - 1) - yi) * Wg
                dz = (jnp.minimum(zi + 1, D - 1) - zi) * (Wg * Hg)
                base3 = (zi * Hg + yi) * Wg + xi
                idx_r[0, sl] = base3
                idx_r[1, sl] = base3 + dy
                idx_r[2, sl] = base3 + dz
                idx_r[3, sl] = base3 + dz + dy
                ox = 1.0 - fx
                oy = 1.0 - fy
                oz = 1.0 - fz
                wzy = (oz * oy, oz * fy, fz * oy, fz * fy)
                for r in range(4):
                    w_r[2 * r, sl] = wzy[r] * ox
                    w_r[2 * r + 1, sl] = wzy[r] * fx
                for k, (a, b) in enumerate(_PLANE_DIMID):
                    PHk, PWk = pdims[k]
                    gu = coords[a]
                    gv = coords[b]
                    iu = (gu + 1.0) * 0.5 * (PWk - 1)
                    iv = (gv + 1.0) * 0.5 * (PHk - 1)
                    ui = jnp.clip(iu.astype(jnp.int32), 0, PWk - 1)
                    vi = jnp.clip(iv.astype(jnp.int32), 0, PHk - 1)
                    fu = iu - ui.astype(jnp.float32)
                    fv = iv - vi.astype(jnp.float32)
                    dv = (jnp.minimum(vi + 1, PHk - 1) - vi) * PWk
                    r0 = vi * PWk + ui
                    t0 = 4 + 2 * k
                    idx_r[t0, sl] = r0
                    idx_r[t0 + 1, sl] = r0 + dv
                    ou = 1.0 - fu
                    ov = 1.0 - fv
                    w_r[8 + 4 * k + 0, sl] = ov * ou
                    w_r[8 + 4 * k + 1, sl] = ov * fu
                    w_r[8 + 4 * k + 2, sl] = fv * ou
                    w_r[8 + 4 * k + 3, sl] = fv * fu
                return c
            return prep_group

        def unpack2(word):
            u = plsc.bitcast(word, jnp.uint32)
            flo = plsc.bitcast(jnp.left_shift(u, jnp.uint32(16)), jnp.float32)
            fhi = plsc.bitcast(u & jnp.uint32(0xFFFF0000), jnp.float32)
            return flo, fhi

        def make_combine(rows_r, w_r):
            def combine_group(g, c):
                sl = pl.ds(g * _L, _L)
                pvec = lane + g * _L
                wvs = [w_r[t, sl] for t in range(2 * _NT)]
                H = C // 2
                for w in range(H):
                    wv0 = jnp.full((_L,), w, jnp.int32)
                    wv1 = jnp.full((_L,), w + H, jnp.int32)
                    alo = None
                    ahi = None
                    for t in range(4):
                        tv = jnp.full((_L,), t, jnp.int32)
                        x0 = plsc.load_gather(rows_r, [tv, pvec, wv0])
                        x1 = plsc.load_gather(rows_r, [tv, pvec, wv1])
                        l0, h0 = unpack2(x0)
                        l1, h1 = unpack2(x1)
                        ul = wvs[2 * t] * l0 + wvs[2 * t + 1] * l1
                        uh = wvs[2 * t] * h0 + wvs[2 * t + 1] * h1
                        alo = ul if alo is None else alo + ul
                        ahi = uh if ahi is None else ahi + uh
                    for k in range(3):
                        mlo = None
                        mhi = None
                        for h in range(2):
                            t = 4 + 2 * k + h
                            tv = jnp.full((_L,), t, jnp.int32)
                            x0 = plsc.load_gather(rows_r, [tv, pvec, wv0])
                            x1 = plsc.load_gather(rows_r, [tv, pvec, wv1])
                            l0, h0 = unpack2(x0)
                            l1, h1 = unpack2(x1)
                            wa = wvs[8 + 4 * k + 2 * h]
                            wb = wvs[8 + 4 * k + 2 * h + 1]
                            ul = wa * l0 + wb * l1
                            uh = wa * h0 + wb * h1
                            mlo = ul if mlo is None else mlo + ul
                            mhi = uh if mhi is None else mhi + uh
                        alo = alo * mlo
                        ahi = ahi * mhi
                    outv[2 * w, sl] = alo
                    outv[2 * w + 1, sl] = ahi
                return c
            return combine_group

        preps = (make_prep(idx0, w0), make_prep(idx1, w1))
        combines = (make_combine(rows0, w0), make_combine(rows1, w1))

        def stage_prep_fire(cc, par):
            base = tile_base + cc * P
            pltpu.sync_copy(xf.at[pl.ds(base * 6, P * 6)], xv)
            lax.fori_loop(0, ngroups, preps[par], 0)
            for t in range(_NT):
                pltpu.async_copy(tabs[t].at[idxs[par].at[t]], rows[par].at[t],
                                 sems[par])

        def wait_combine_store(cc, par):
            for t in range(_NT):
                pltpu.make_async_copy(tabs[t].at[idxs[par].at[t]],
                                      rows[par].at[t], sems[par]).wait()
            lax.fori_loop(0, ngroups, combines[par], 0)
            base = tile_base + cc * P
            pltpu.sync_copy(outv, out.at[:, pl.ds(base, P)])

        if pipelined:
            stage_prep_fire(0, 0)

            def pair(j, c):
                c0 = 2 * j
                stage_prep_fire(c0 + 1, 1)
                wait_combine_store(c0, 0)

                @pl.when(c0 + 2 < nchunks)
                def _fire_next():
                    stage_prep_fire(c0 + 2, 0)

                wait_combine_store(c0 + 1, 1)
                return c

            lax.fori_loop(0, nchunks // 2, pair, 0)
        else:
            def chunk(kk, c):
                stage_prep_fire(kk, 0)
                wait_combine_store(kk, 0)
                return c

            lax.fori_loop(0, nchunks, chunk, 0)

    return pl.kernel(body, out_type=out_type, mesh=mesh,
                     scratch_types=scratch, interpret=interpret,
                     compiler_params=pltpu.CompilerParams(
                         needs_layout_passes=False,
                         use_tc_tiling_on_sc=False))


def kernel(x, feature_grid_3d, plane0, plane1, plane2):
    B = x.shape[0]
    C = feature_grid_3d.shape[1]
    D, Hg, Wg = feature_grid_3d.shape[2:5]
    assert C == _L and B % (_NW * _L) == 0
    pdims = [(p.shape[2], p.shape[3]) for p in (plane0, plane1, plane2)]
    sizes = [D * Hg * Wg] + [ph * pw for ph, pw in pdims]
    rk = _make_relayout_kernel(C, tuple(sizes))
    flats = rk(feature_grid_3d.reshape(C, -1),
               plane0.reshape(C, -1),
               plane1.reshape(C, -1),
               plane2.reshape(C, -1))
    g3t, p0t, p1t, p2t = (f.reshape(n, C) for f, n in zip(flats, sizes))
    k = _make_sc_kernel(B, C, (D, Hg, Wg), tuple(pdims))
    return k(x.reshape(-1), g3t, p0t, p1t, p2t)
